# baseline (speedup 1.0000x reference)
import functools

import jax
import jax.numpy as jnp
from jax import lax
from jax.experimental import pallas as pl
from jax.experimental.pallas import tpu as pltpu
from jax.experimental.pallas import tpu_sc as plsc

_NC = 2
_NS = 16
_NW = _NC * _NS

_EDGE_CHUNK = 80
_NBUF = 4
_IRING = 8


def _make_aggregate(n, e, d):
    assert e % _NW == 0
    edges_per_worker = e // _NW
    assert edges_per_worker % _EDGE_CHUNK == 0
    chunks = edges_per_worker // _EDGE_CHUNK
    rows_per_tile = (n // _NS) // 8 * 8
    tail_rows = n - _NS * rows_per_tile
    assert tail_rows % 8 == 0 and tail_rows <= rows_per_tile

    mesh = plsc.VectorSubcoreMesh(core_axis_name="c", subcore_axis_name="s")

    assert chunks >= _IRING

    @functools.partial(
        pl.kernel,
        out_type=jax.ShapeDtypeStruct((_NC * n, d), jnp.float32),
        mesh=mesh,
        scratch_types=[pltpu.VMEM((_NBUF, _EDGE_CHUNK, d), jnp.float32)]
        + [pltpu.VMEM((_EDGE_CHUNK,), jnp.int32)] * (2 * _IRING)
        + [pltpu.VMEM_SHARED((n, d), jnp.float32)]
        + [pltpu.SemaphoreType.DMA] * (2 * _NBUF + 2 * _IRING),
    )
    def agg_kernel(x_hbm, src_hbm, dst_hbm, out_hbm, rows, *rest):
        sidx = rest[:_IRING]
        didx = rest[_IRING:2 * _IRING]
        acc_sh = rest[2 * _IRING]
        sems = rest[2 * _IRING + 1:]
        gsem = sems[:_NBUF]
        ssem = sems[_NBUF:2 * _NBUF]
        issem = sems[2 * _NBUF:2 * _NBUF + _IRING]
        idsem = sems[2 * _NBUF + _IRING:]
        cid = lax.axis_index("c")
        sid = lax.axis_index("s")
        wid = sid * _NC + cid
        row0 = pl.multiple_of(sid * rows_per_tile, 8)

        def zbody(i, c):
            for jj in range(d // 16):
                rows[0, i, pl.ds(jj * 16, 16)] = jnp.zeros((16,), jnp.float32)
            return c

        lax.fori_loop(0, _EDGE_CHUNK, zbody, 0)
        full, rem = divmod(rows_per_tile, _EDGE_CHUNK)
        for k in range(full):
            pltpu.sync_copy(rows.at[0],
                            acc_sh.at[pl.ds(row0 + k * _EDGE_CHUNK, _EDGE_CHUNK)])
        if rem:
            pltpu.sync_copy(rows.at[0, pl.ds(0, rem)],
                            acc_sh.at[pl.ds(row0 + full * _EDGE_CHUNK, rem)])
        if tail_rows:
            @pl.when(sid == 0)
            def _zero_tail():
                pltpu.sync_copy(rows.at[0, pl.ds(0, tail_rows)],
                                acc_sh.at[pl.ds(_NS * rows_per_tile, tail_rows)])
        plsc.subcore_barrier()

        ebase = wid * edges_per_worker

        def fetch_idx(j, slot):
            base = ebase + j * _EDGE_CHUNK
            pltpu.async_copy(src_hbm.at[pl.ds(base, _EDGE_CHUNK)],
                             sidx[slot], issem[slot])
            pltpu.async_copy(dst_hbm.at[pl.ds(base, _EDGE_CHUNK)],
                             didx[slot], idsem[slot])

        def wait_sidx(j, slot):
            base = ebase + j * _EDGE_CHUNK
            pltpu.make_async_copy(src_hbm.at[pl.ds(base, _EDGE_CHUNK)],
                                  sidx[slot], issem[slot]).wait()

        def wait_didx(j, slot):
            base = ebase + j * _EDGE_CHUNK
            pltpu.make_async_copy(dst_hbm.at[pl.ds(base, _EDGE_CHUNK)],
                                  didx[slot], idsem[slot]).wait()

        for b in range(_IRING):
            fetch_idx(b, b)
        for b in range(_NBUF):
            wait_sidx(b, b)
            pltpu.async_copy(x_hbm.at[sidx[b]], rows.at[b], gsem[b])

        rounds = (chunks + _IRING - 1) // _IRING

        def body(g, carry):
            for u in range(_IRING):
                j = g * _IRING + u
                br = u % _NBUF

                @pl.when(j < chunks)
                def _step(j=j, u=u, br=br):
                    pltpu.make_async_copy(x_hbm.at[sidx[u]], rows.at[br],
                                          gsem[br]).wait()
                    wait_didx(j, u)
                    pltpu.async_copy(rows.at[br], acc_sh.at[didx[u]],
                                     ssem[br], add=True).wait()

                    @pl.when(j + _IRING < chunks)
                    def _fetch(j=j, u=u):
                        fetch_idx(j + _IRING, u)

                    @pl.when(j + _NBUF < chunks)
                    def _gather(j=j, u=u, br=br):
                        un = (u + _NBUF) % _IRING
                        wait_sidx(j + _NBUF, un)
                        pltpu.async_copy(x_hbm.at[sidx[un]], rows.at[br],
                                         gsem[br])
            return carry

        lax.fori_loop(0, rounds, body, 0)
        plsc.subcore_barrier()
        out0 = pl.multiple_of(cid * n + row0, 8)
        pltpu.sync_copy(acc_sh.at[pl.ds(row0, rows_per_tile)],
                        out_hbm.at[pl.ds(out0, rows_per_tile)])
        if tail_rows:
            @pl.when(sid == 0)
            def _copy_tail():
                tbase = _NS * rows_per_tile
                tout = pl.multiple_of(cid * n + tbase, 8)
                pltpu.sync_copy(acc_sh.at[pl.ds(tbase, tail_rows)],
                                out_hbm.at[pl.ds(tout, tail_rows)])

    return agg_kernel


_BLK = 10000


def _mlp1_body(x_ref, pa_ref, pb_ref, wa, ba, wb, bb, o_ref):
    h = x_ref[...] + pa_ref[...] + pb_ref[...]
    h = jnp.maximum(jnp.dot(h, wa[...], preferred_element_type=jnp.float32) + ba[...], 0.0)
    h = jnp.dot(h, wb[...], preferred_element_type=jnp.float32) + bb[...]
    o_ref[...] = jnp.maximum(h, 0.0)


def _mlp2_body(x_ref, pa_ref, pb_ref, wa, ba, wb, bb, wc, bc, o_ref):
    h = x_ref[...] + pa_ref[...] + pb_ref[...]
    h = jnp.maximum(jnp.dot(h, wa[...], preferred_element_type=jnp.float32) + ba[...], 0.0)
    h = jnp.dot(h, wb[...], preferred_element_type=jnp.float32) + bb[...]
    o_ref[...] = jnp.dot(h, wc[...], preferred_element_type=jnp.float32) + bc[...]


def _row_specs(n, d):
    nblk = n // _BLK
    row = pl.BlockSpec((_BLK, d), lambda i: (i, 0))
    pa = pl.BlockSpec((_BLK, d), lambda i: (i, 0))
    pb = pl.BlockSpec((_BLK, d), lambda i, _nb=nblk: (i + _nb, 0))
    w = pl.BlockSpec((d, d), lambda i: (0, 0))
    b = pl.BlockSpec((1, d), lambda i: (0, 0))
    return nblk, row, pa, pb, w, b


def _mlp1(x, p, wa, ba, wb, bb):
    n, d = x.shape
    nblk, row, pa, pb, w, b = _row_specs(n, d)
    return pl.pallas_call(
        _mlp1_body,
        grid=(nblk,),
        in_specs=[row, pa, pb, w, b, w, b],
        out_specs=row,
        out_shape=jax.ShapeDtypeStruct((n, d), jnp.float32),
    )(x, p, p, wa, ba.reshape(1, d), wb, bb.reshape(1, d))


def _mlp2(x, p, wa, ba, wb, bb, wc, bc):
    n, d = x.shape
    nblk, row, pa, pb, w, b = _row_specs(n, d)
    return pl.pallas_call(
        _mlp2_body,
        grid=(nblk,),
        in_specs=[row, pa, pb, w, b, w, b, w, b],
        out_specs=row,
        out_shape=jax.ShapeDtypeStruct((n, d), jnp.float32),
    )(x, p, p, wa, ba.reshape(1, d), wb, bb.reshape(1, d), wc, bc.reshape(1, d))


def kernel(x, edge_index, W1a, b1a, W1b, b1b, W2a, b2a, W2b, b2b, Wfc, bfc):
    n, d = x.shape
    e = edge_index.shape[1]
    src = edge_index[0].astype(jnp.int32)
    dst = edge_index[1].astype(jnp.int32)
    agg = _make_aggregate(n, e, d)
    p1 = agg(x, src, dst)
    h1 = _mlp1(x, p1, W1a, b1a, W1b, b1b)
    p2 = agg(h1, src, dst)
    return _mlp2(h1, p2, W2a, b2a, W2b, b2b, Wfc, bfc)


# speedup vs baseline: 1.0514x; 1.0514x over previous
import functools

import jax
import jax.numpy as jnp
from jax import lax
from jax.experimental import pallas as pl
from jax.experimental.pallas import tpu as pltpu
from jax.experimental.pallas import tpu_sc as plsc

_NC = 2
_NS = 16
_NW = _NC * _NS

_EDGE_CHUNK = 80
_NBUF = 4
_IRING = 8


def _make_aggregate(n, e, d):
    assert e % _NW == 0
    edges_per_worker = e // _NW
    assert edges_per_worker % _EDGE_CHUNK == 0
    chunks = edges_per_worker // _EDGE_CHUNK
    rows_per_tile = (n // _NS) // 8 * 8
    tail_rows = n - _NS * rows_per_tile
    assert tail_rows % 8 == 0 and tail_rows <= rows_per_tile

    mesh = plsc.VectorSubcoreMesh(core_axis_name="c", subcore_axis_name="s")

    assert chunks >= _IRING

    @functools.partial(
        pl.kernel,
        out_type=jax.ShapeDtypeStruct((_NC * n, d), jnp.float32),
        mesh=mesh,
        scratch_types=[pltpu.VMEM((_NBUF, _EDGE_CHUNK, d), jnp.float32)]
        + [pltpu.VMEM((_EDGE_CHUNK,), jnp.int32)] * (2 * _IRING)
        + [pltpu.VMEM_SHARED((n, d), jnp.float32)]
        + [pltpu.SemaphoreType.DMA] * (2 * _NBUF + 2 * _IRING),
    )
    def agg_kernel(x_hbm, ei_hbm, out_hbm, rows, *rest):
        sidx = rest[:_IRING]
        didx = rest[_IRING:2 * _IRING]
        acc_sh = rest[2 * _IRING]
        sems = rest[2 * _IRING + 1:]
        gsem = sems[:_NBUF]
        ssem = sems[_NBUF:2 * _NBUF]
        issem = sems[2 * _NBUF:2 * _NBUF + _IRING]
        idsem = sems[2 * _NBUF + _IRING:]
        cid = lax.axis_index("c")
        sid = lax.axis_index("s")
        wid = sid * _NC + cid
        row0 = pl.multiple_of(sid * rows_per_tile, 8)

        def zbody(i, c):
            for jj in range(d // 16):
                rows[0, i, pl.ds(jj * 16, 16)] = jnp.zeros((16,), jnp.float32)
            return c

        lax.fori_loop(0, _EDGE_CHUNK, zbody, 0)
        full, rem = divmod(rows_per_tile, _EDGE_CHUNK)
        for k in range(full):
            pltpu.sync_copy(rows.at[0],
                            acc_sh.at[pl.ds(row0 + k * _EDGE_CHUNK, _EDGE_CHUNK)])
        if rem:
            pltpu.sync_copy(rows.at[0, pl.ds(0, rem)],
                            acc_sh.at[pl.ds(row0 + full * _EDGE_CHUNK, rem)])
        if tail_rows:
            @pl.when(sid == 0)
            def _zero_tail():
                pltpu.sync_copy(rows.at[0, pl.ds(0, tail_rows)],
                                acc_sh.at[pl.ds(_NS * rows_per_tile, tail_rows)])
        plsc.subcore_barrier()

        ebase = wid * edges_per_worker

        def fetch_idx(j, slot):
            base = ebase + j * _EDGE_CHUNK
            pltpu.async_copy(ei_hbm.at[pl.ds(base, _EDGE_CHUNK)],
                             sidx[slot], issem[slot])
            pltpu.async_copy(ei_hbm.at[pl.ds(e + base, _EDGE_CHUNK)],
                             didx[slot], idsem[slot])

        def wait_sidx(j, slot):
            base = ebase + j * _EDGE_CHUNK
            pltpu.make_async_copy(ei_hbm.at[pl.ds(base, _EDGE_CHUNK)],
                                  sidx[slot], issem[slot]).wait()

        def wait_didx(j, slot):
            base = ebase + j * _EDGE_CHUNK
            pltpu.make_async_copy(ei_hbm.at[pl.ds(e + base, _EDGE_CHUNK)],
                                  didx[slot], idsem[slot]).wait()

        for b in range(_IRING):
            fetch_idx(b, b)
        for b in range(_NBUF):
            wait_sidx(b, b)
            pltpu.async_copy(x_hbm.at[sidx[b]], rows.at[b], gsem[b])

        rounds = (chunks + _IRING - 1) // _IRING

        def body(g, carry):
            for u in range(_IRING):
                j = g * _IRING + u
                br = u % _NBUF

                @pl.when(j < chunks)
                def _step(j=j, u=u, br=br):
                    pltpu.make_async_copy(x_hbm.at[sidx[u]], rows.at[br],
                                          gsem[br]).wait()
                    wait_didx(j, u)
                    pltpu.async_copy(rows.at[br], acc_sh.at[didx[u]],
                                     ssem[br], add=True).wait()

                    @pl.when(j + _IRING < chunks)
                    def _fetch(j=j, u=u):
                        fetch_idx(j + _IRING, u)

                    @pl.when(j + _NBUF < chunks)
                    def _gather(j=j, u=u, br=br):
                        un = (u + _NBUF) % _IRING
                        wait_sidx(j + _NBUF, un)
                        pltpu.async_copy(x_hbm.at[sidx[un]], rows.at[br],
                                         gsem[br])
            return carry

        lax.fori_loop(0, rounds, body, 0)
        plsc.subcore_barrier()
        out0 = pl.multiple_of(cid * n + row0, 8)
        pltpu.sync_copy(acc_sh.at[pl.ds(row0, rows_per_tile)],
                        out_hbm.at[pl.ds(out0, rows_per_tile)])
        if tail_rows:
            @pl.when(sid == 0)
            def _copy_tail():
                tbase = _NS * rows_per_tile
                tout = pl.multiple_of(cid * n + tbase, 8)
                pltpu.sync_copy(acc_sh.at[pl.ds(tbase, tail_rows)],
                                out_hbm.at[pl.ds(tout, tail_rows)])

    return agg_kernel


_BLK = 5000


def _mlp1_body(x_ref, pa_ref, pb_ref, wa, ba, wb, bb, o_ref):
    h = x_ref[...] + pa_ref[...] + pb_ref[...]
    h = jnp.maximum(jnp.dot(h, wa[...], preferred_element_type=jnp.float32) + ba[...], 0.0)
    h = jnp.dot(h, wb[...], preferred_element_type=jnp.float32) + bb[...]
    o_ref[...] = jnp.maximum(h, 0.0)


def _mlp2_body(x_ref, pa_ref, pb_ref, wa, ba, wb, bb, wc, bc, o_ref):
    h = x_ref[...] + pa_ref[...] + pb_ref[...]
    h = jnp.maximum(jnp.dot(h, wa[...], preferred_element_type=jnp.float32) + ba[...], 0.0)
    h = jnp.dot(h, wb[...], preferred_element_type=jnp.float32) + bb[...]
    o_ref[...] = jnp.dot(h, wc[...], preferred_element_type=jnp.float32) + bc[...]


def _row_specs(n, d):
    nblk = n // _BLK
    row = pl.BlockSpec((_BLK, d), lambda i: (i, 0))
    pa = pl.BlockSpec((_BLK, d), lambda i: (i, 0))
    pb = pl.BlockSpec((_BLK, d), lambda i, _nb=nblk: (i + _nb, 0))
    w = pl.BlockSpec((d, d), lambda i: (0, 0))
    b = pl.BlockSpec((1, d), lambda i: (0, 0))
    return nblk, row, pa, pb, w, b


def _mlp1(x, p, wa, ba, wb, bb):
    n, d = x.shape
    nblk, row, pa, pb, w, b = _row_specs(n, d)
    return pl.pallas_call(
        _mlp1_body,
        grid=(nblk,),
        in_specs=[row, pa, pb, w, b, w, b],
        out_specs=row,
        out_shape=jax.ShapeDtypeStruct((n, d), jnp.float32),
    )(x, p, p, wa, ba.reshape(1, d), wb, bb.reshape(1, d))


def _mlp2(x, p, wa, ba, wb, bb, wc, bc):
    n, d = x.shape
    nblk, row, pa, pb, w, b = _row_specs(n, d)
    return pl.pallas_call(
        _mlp2_body,
        grid=(nblk,),
        in_specs=[row, pa, pb, w, b, w, b, w, b],
        out_specs=row,
        out_shape=jax.ShapeDtypeStruct((n, d), jnp.float32),
    )(x, p, p, wa, ba.reshape(1, d), wb, bb.reshape(1, d), wc, bc.reshape(1, d))


def kernel(x, edge_index, W1a, b1a, W1b, b1b, W2a, b2a, W2b, b2b, Wfc, bfc):
    n, d = x.shape
    e = edge_index.shape[1]
    ei = edge_index.astype(jnp.int32).reshape(2 * e)
    agg = _make_aggregate(n, e, d)
    p1 = agg(x, ei)
    h1 = _mlp1(x, p1, W1a, b1a, W1b, b1b)
    p2 = agg(h1, ei)
    return _mlp2(h1, p2, W2a, b2a, W2b, b2b, Wfc, bfc)
